# baseline (speedup 1.0000x reference)
import functools

import jax
import jax.numpy as jnp
from jax import lax
from jax.experimental import pallas as pl
from jax.experimental.pallas import tpu as pltpu
from jax.experimental.pallas import tpu_sc as plsc

_N = 8 * 96 * 224 * 224
_NW = 32
_CH = 8192
_NV = _CH // 16
_NCH = _N // (_NW * _CH)
_PER_W = _NCH * _CH


def _sc_body(rec_hbm, tgt_hbm, out_hbm, bufr, buft, outbuf, sem0, sem1):
    wid = lax.axis_index("s") * 2 + lax.axis_index("c")
    base = wid * _PER_W

    def start(k, slot_r, slot_t, sem):
        pltpu.async_copy(rec_hbm.at[pl.ds(base + k * _CH, _CH)], slot_r, sem)
        pltpu.async_copy(tgt_hbm.at[pl.ds(base + k * _CH, _CH)], slot_t, sem)

    def drain(slot_r, slot_t, sem):
        pltpu.make_async_copy(rec_hbm.at[pl.ds(base, _CH)], slot_r, sem).wait()
        pltpu.make_async_copy(tgt_hbm.at[pl.ds(base, _CH)], slot_t, sem).wait()

    def lane_acc(a, r, t):
        s_all, s_z, s_fn, c_z, c_rz, c_rt, c_tn = a
        d = r - t
        sq = d * d
        zm = t == 0.0
        rz = r == 0.0
        rt = r == t
        tn = zm & rz
        fn = tn != rz
        zf = jnp.zeros((16,), jnp.float32)
        zi = jnp.zeros((16,), jnp.int32)
        oi = jnp.ones((16,), jnp.int32)
        return (s_all + sq,
                s_z + jnp.where(zm, sq, zf),
                s_fn + jnp.where(fn, sq, zf),
                c_z + jnp.where(zm, oi, zi),
                c_rz + jnp.where(rz, oi, zi),
                c_rt + jnp.where(rt, oi, zi),
                c_tn + jnp.where(tn, oi, zi))

    def chunk_acc(slot_r, slot_t, acc):
        def step(i, ab):
            a, b = ab
            a = lane_acc(a, slot_r[pl.ds(i * 32, 16)],
                         slot_t[pl.ds(i * 32, 16)])
            b = lane_acc(b, slot_r[pl.ds(i * 32 + 16, 16)],
                         slot_t[pl.ds(i * 32 + 16, 16)])
            return (a, b)
        return lax.fori_loop(0, _NV // 2, step, acc)

    def zacc():
        return (tuple(jnp.zeros((16,), jnp.float32) for _ in range(3))
                + tuple(jnp.zeros((16,), jnp.int32) for _ in range(4)))

    acc0 = (zacc(), zacc())

    start(0, bufr.at[0], buft.at[0], sem0)

    def outer(i, acc):
        k = i * 2
        start(k + 1, bufr.at[1], buft.at[1], sem1)
        drain(bufr.at[0], buft.at[0], sem0)
        acc = chunk_acc(bufr.at[0], buft.at[0], acc)
        start(k + 2, bufr.at[0], buft.at[0], sem0)
        drain(bufr.at[1], buft.at[1], sem1)
        return chunk_acc(bufr.at[1], buft.at[1], acc)

    acc = lax.fori_loop(0, (_NCH - 1) // 2, outer, acc0)
    drain(bufr.at[0], buft.at[0], sem0)
    acc = chunk_acc(bufr.at[0], buft.at[0], acc)
    a, b = acc

    for i in range(3):
        outbuf[i, :] = a[i] + b[i]
    for i in range(3, 7):
        outbuf[i, :] = (a[i] + b[i]).astype(jnp.float32)
    pltpu.sync_copy(outbuf, out_hbm.at[wid])


_sc_call_cache = []


def _sc_call(rec_flat, tgt_flat):
    if not _sc_call_cache:
        _sc_call_cache.append(functools.partial(
            pl.kernel,
            out_type=jax.ShapeDtypeStruct((_NW, 7, 16), jnp.float32),
            mesh=plsc.VectorSubcoreMesh(core_axis_name="c",
                                        subcore_axis_name="s"),
            scratch_types=[
                pltpu.VMEM((2, _CH), jnp.float32),
                pltpu.VMEM((2, _CH), jnp.float32),
                pltpu.VMEM((7, 16), jnp.float32),
                pltpu.SemaphoreType.DMA,
                pltpu.SemaphoreType.DMA,
            ],
        )(_sc_body))
    return _sc_call_cache[0](rec_flat, tgt_flat)


def _combine_body(parts_ref, out_ref):
    p = parts_ref[...]
    s_all = jnp.sum(p[:, 0, :])
    s_z = jnp.sum(p[:, 1, :])
    s_fn = jnp.sum(p[:, 2, :])
    c_z = jnp.sum(p[:, 3, :].astype(jnp.int32))
    c_rz = jnp.sum(p[:, 4, :].astype(jnp.int32))
    c_rt = jnp.sum(p[:, 5, :].astype(jnp.int32))
    c_tn = jnp.sum(p[:, 6, :].astype(jnp.int32))
    c_fn = c_rz - c_tn
    c_tm = c_rt - c_tn

    n_f = jnp.float32(_N)
    c_nz = _N - c_z
    s_nz = s_all - s_z
    c_tp = c_nz - c_fn
    s_tp = s_nz - s_fn
    c_fp = c_z - c_tn

    def mse(s, c, repl):
        m = s / jnp.maximum(c, 1).astype(jnp.float32)
        return jnp.where(c == 0, jnp.float32(repl), m)

    ff_loss = s_all / n_f
    zero_loss = mse(s_z, c_z, 0.0)
    nonzero_loss = mse(s_nz, c_nz, 0.0)
    time_match = jnp.where(c_tm == 0, jnp.float32(10.0), jnp.float32(0.0))
    fnl = mse(s_fn, c_fn, 0.0)
    fpl = mse(s_tp, c_tp, 0.0)
    tnl = jnp.where(c_tn == 0, jnp.float32(10.0), jnp.float32(0.0))
    tpl = mse(s_z, c_fp, 10.0)
    out_ref[0, 0] = (tpl + fnl + fpl + tnl + time_match
                     + ff_loss + zero_loss + nonzero_loss)


def _combine(parts, interpret=False):
    return pl.pallas_call(
        _combine_body,
        out_specs=pl.BlockSpec(memory_space=pltpu.SMEM),
        out_shape=jax.ShapeDtypeStruct((1, 1), jnp.float32),
        interpret=interpret,
    )(parts)


def kernel(reconstructed_image, target_image):
    rec_flat = reconstructed_image.reshape(_N)
    tgt_flat = target_image.reshape(_N)
    parts = _sc_call(rec_flat, tgt_flat)
    return _combine(parts)[0, 0]
